# baseline (speedup 1.0000x reference)
import functools

import jax
import jax.numpy as jnp
from jax import lax
from jax.experimental import pallas as pl
from jax.experimental.pallas import tpu as pltpu
from jax.experimental.pallas import tpu_sc as plsc

N = 10000
E = 320000
C = 128
K = 128
NC = 2
NS = 16
NW = NC * NS
BPW = 80
EP = NW * BPW * K
NSLOT = 4
ROWS_PER_TILE = 624


def _spmm_body(scale, x_hbm, src_hbm, dst_hbm, w_hbm, part_hbm,
               acc, idx_v, w_v, rows,
               gs0, gs1, ss0, ss1, is0, is1, is2, is3):
    gsems = (gs0, gs1)
    ssems = (ss0, ss1)
    isems = (is0, is1, is2, is3)
    cid = lax.axis_index("c")
    sid = lax.axis_index("s")
    wid = sid * NC + cid
    ebase = wid * (BPW * K)

    def issue_idx(blk, slot):
        off = ebase + blk * K
        pltpu.async_copy(src_hbm.at[pl.ds(off, K)], idx_v.at[slot, 0],
                         isems[slot])
        pltpu.async_copy(dst_hbm.at[pl.ds(off, K)], idx_v.at[slot, 1],
                         isems[slot])
        pltpu.async_copy(w_hbm.at[pl.ds(off, K)], w_v.at[slot], isems[slot])

    def wait_idx(blk, slot):
        off = ebase + blk * K
        pltpu.make_async_copy(src_hbm.at[pl.ds(off, K)], idx_v.at[slot, 0],
                              isems[slot]).wait()
        pltpu.make_async_copy(dst_hbm.at[pl.ds(off, K)], idx_v.at[slot, 1],
                              isems[slot]).wait()
        pltpu.make_async_copy(w_hbm.at[pl.ds(off, K)], w_v.at[slot],
                              isems[slot]).wait()

    issue_idx(0, 0)
    wait_idx(0, 0)
    issue_idx(1, 1)

    def zbody(r, _):
        for v in range(C // 16):
            rows[0, r, pl.ds(16 * v, 16)] = jnp.zeros((16,), jnp.float32)
        return 0
    lax.fori_loop(0, K, zbody, 0)
    base = sid * ROWS_PER_TILE
    for q in range(ROWS_PER_TILE // K):
        pltpu.sync_copy(rows.at[0], acc.at[pl.ds(base + q * K, K)])
    rem = ROWS_PER_TILE % K
    pltpu.sync_copy(rows.at[0, pl.ds(0, rem)],
                    acc.at[pl.ds(base + ROWS_PER_TILE - rem, rem)])

    @pl.when(sid == NS - 1)
    def _():
        tail = NS * ROWS_PER_TILE
        pltpu.sync_copy(rows.at[0, pl.ds(0, 16)],
                        acc.at[pl.ds(tail, N - tail)])

    pltpu.async_copy(x_hbm.at[idx_v.at[0, 0]], rows.at[0], gsems[0])

    plsc.subcore_barrier()

    NIT = BPW // 4

    def stage(j, u, i):
        p = u % 2
        q = 1 - p
        t1 = (u + 1) % 4
        t2 = (u + 2) % 4

        pltpu.make_async_copy(x_hbm.at[idx_v.at[u, 0]],
                              rows.at[p], gsems[p]).wait()

        def step2():
            def wait_sc():
                pltpu.make_async_copy(
                    rows.at[q], acc.at[idx_v.at[(u + 3) % 4, 1]],
                    ssems[q]).wait()
            if u == 0:
                pl.when(i >= 1)(wait_sc)
            else:
                wait_sc()

            def pref():
                issue_idx(j + 2, t2)
            if u < 2:
                pref()
            else:
                pl.when(i < NIT - 1)(pref)

            wait_idx(j + 1, t1)
            pltpu.async_copy(x_hbm.at[idx_v.at[t1, 0]], rows.at[q],
                             gsems[q])
        if u == 3:
            pl.when(i < NIT - 1)(step2)
        else:
            step2()

        def mbody(h, _):
            w16 = w_v[u, pl.ds(h * 16, 16)] * scale
            for l in range(16):
                ws = w16[l]
                e = h * 16 + l
                for v in range(C // 16):
                    rows[p, e, pl.ds(16 * v, 16)] = (
                        rows[p, e, pl.ds(16 * v, 16)] * ws)
            return 0
        lax.fori_loop(0, K // 16, mbody, 0)

        pltpu.async_copy(rows.at[p], acc.at[idx_v.at[u, 1]],
                         ssems[p], add=True)

    def it(i, _):
        for u in range(4):
            stage(4 * i + u, u, i)
        return 0
    lax.fori_loop(0, NIT, it, 0)

    pltpu.make_async_copy(rows.at[0], acc.at[idx_v.at[2, 1]],
                          ssems[0]).wait()
    pltpu.make_async_copy(rows.at[1], acc.at[idx_v.at[3, 1]],
                          ssems[1]).wait()

    plsc.subcore_barrier()
    pltpu.sync_copy(acc.at[pl.ds(base, ROWS_PER_TILE)],
                    part_hbm.at[cid, pl.ds(base, ROWS_PER_TILE)])

    @pl.when(sid == NS - 1)
    def _():
        tail = NS * ROWS_PER_TILE
        pltpu.sync_copy(acc.at[pl.ds(tail, N - tail)],
                        part_hbm.at[cid, pl.ds(tail, N - tail)])


@functools.lru_cache(maxsize=None)
def _make_spmm(scale):
    mesh = plsc.VectorSubcoreMesh(core_axis_name="c", subcore_axis_name="s")
    return pl.kernel(
        functools.partial(_spmm_body, scale),
        out_type=jax.ShapeDtypeStruct((NC, N, C), jnp.float32),
        mesh=mesh,
        scratch_types=[
            pltpu.VMEM_SHARED((N, C), jnp.float32),
            pltpu.VMEM((4, 2, K), jnp.int32),
            pltpu.VMEM((4, K), jnp.float32),
            pltpu.VMEM((2, K, C), jnp.float32),
        ] + [pltpu.SemaphoreType.DMA] * 8,
    )


_ROWS_BLK = 1000
_GRID = N // _ROWS_BLK


def _combine_body(p0_ref, p1_ref, prev_ref, o_ref):
    o_ref[...] = p0_ref[...] + p1_ref[...] - prev_ref[...]


_combine = pl.pallas_call(
    _combine_body,
    grid=(_GRID,),
    in_specs=[pl.BlockSpec((_ROWS_BLK, C), lambda i: (i, 0))] * 3,
    out_specs=pl.BlockSpec((_ROWS_BLK, C), lambda i: (i, 0)),
    out_shape=jax.ShapeDtypeStruct((N, C), jnp.float32),
)


def _combine2_body(p0_ref, p1_ref, o_ref):
    o_ref[...] = p0_ref[...] + p1_ref[...]


_combine2 = pl.pallas_call(
    _combine2_body,
    grid=(_GRID,),
    in_specs=[pl.BlockSpec((_ROWS_BLK, C), lambda i: (i, 0))] * 2,
    out_specs=pl.BlockSpec((_ROWS_BLK, C), lambda i: (i, 0)),
    out_shape=jax.ShapeDtypeStruct((N, C), jnp.float32),
)


def _final_body(c0_ref, c1_ref, c2_ref, c3_ref, w_ref, b_ref, o_ref):
    acc = jnp.dot(c0_ref[...], w_ref[0], preferred_element_type=jnp.float32)
    acc += jnp.dot(c1_ref[...], w_ref[1], preferred_element_type=jnp.float32)
    acc += jnp.dot(c2_ref[...], w_ref[2], preferred_element_type=jnp.float32)
    acc += jnp.dot(c3_ref[...], w_ref[3], preferred_element_type=jnp.float32)
    o_ref[...] = jax.nn.relu(acc + b_ref[...])


_final = pl.pallas_call(
    _final_body,
    grid=(_GRID,),
    in_specs=[pl.BlockSpec((_ROWS_BLK, C), lambda i: (i, 0))] * 4
    + [pl.BlockSpec((4, C, C), lambda i: (0, 0, 0)),
       pl.BlockSpec((1, C), lambda i: (0, 0))],
    out_specs=pl.BlockSpec((_ROWS_BLK, C), lambda i: (i, 0)),
    out_shape=jax.ShapeDtypeStruct((N, C), jnp.float32),
)


def kernel(inputs, edge_index, edge_weight, W, b):
    x = inputs[0]
    pad = EP - E
    src = jnp.pad(edge_index[1], (0, pad))
    dst = jnp.pad(edge_index[0], (0, pad))
    wgt = jnp.pad(edge_weight, (0, pad))

    _spmm_1 = _make_spmm(1.0)
    _spmm_2 = _make_spmm(2.0)

    p1 = _spmm_1(x, src, dst, wgt)
    c1 = _combine2(p1[0], p1[1])
    p2 = _spmm_2(c1, src, dst, wgt)
    c2 = _combine(p2[0], p2[1], x)
    p3 = _spmm_2(c2, src, dst, wgt)
    c3 = _combine(p3[0], p3[1], c1)

    out = _final(x, c1, c2, c3, W, b.reshape(1, C))
    return out[None]


# speedup vs baseline: 3.1079x; 3.1079x over previous
import functools

import jax
import jax.numpy as jnp
from jax import lax
from jax.experimental import pallas as pl
from jax.experimental.pallas import tpu as pltpu
from jax.experimental.pallas import tpu_sc as plsc

N = 10000
E = 320000
C = 128
K = 128
NC = 2
NS = 16
NW = NC * NS
BPW = 80
EP = NW * BPW * K
NSLOT = 4
ROWS_PER_TILE = 624


def _spmm_body(scale, x_hbm, src_hbm, dst_hbm, w_hbm, part_hbm,
               acc, idx_v, w_v, rows,
               gs0, gs1, ss0, ss1, is0, is1, is2, is3):
    gsems = (gs0, gs1)
    ssems = (ss0, ss1)
    isems = (is0, is1, is2, is3)
    cid = lax.axis_index("c")
    sid = lax.axis_index("s")
    wid = sid * NC + cid
    ebase = wid * (BPW * K)

    def issue_idx(blk, slot):
        off = ebase + blk * K
        pltpu.async_copy(src_hbm.at[pl.ds(off, K)], idx_v.at[slot, 0],
                         isems[slot])
        pltpu.async_copy(dst_hbm.at[pl.ds(off, K)], idx_v.at[slot, 1],
                         isems[slot])
        pltpu.async_copy(w_hbm.at[pl.ds(off, K)], w_v.at[slot], isems[slot])

    def wait_idx(blk, slot):
        off = ebase + blk * K
        pltpu.make_async_copy(src_hbm.at[pl.ds(off, K)], idx_v.at[slot, 0],
                              isems[slot]).wait()
        pltpu.make_async_copy(dst_hbm.at[pl.ds(off, K)], idx_v.at[slot, 1],
                              isems[slot]).wait()
        pltpu.make_async_copy(w_hbm.at[pl.ds(off, K)], w_v.at[slot],
                              isems[slot]).wait()

    issue_idx(0, 0)
    wait_idx(0, 0)
    issue_idx(1, 1)

    def zbody(r, _):
        for v in range(C // 16):
            rows[0, r, pl.ds(16 * v, 16)] = jnp.zeros((16,), jnp.float32)
        return 0
    lax.fori_loop(0, K, zbody, 0)
    base = sid * ROWS_PER_TILE
    for q in range(ROWS_PER_TILE // K):
        pltpu.sync_copy(rows.at[0], acc.at[pl.ds(base + q * K, K)])
    rem = ROWS_PER_TILE % K
    pltpu.sync_copy(rows.at[0, pl.ds(0, rem)],
                    acc.at[pl.ds(base + ROWS_PER_TILE - rem, rem)])

    @pl.when(sid == NS - 1)
    def _():
        tail = NS * ROWS_PER_TILE
        pltpu.sync_copy(rows.at[0, pl.ds(0, 16)],
                        acc.at[pl.ds(tail, N - tail)])

    pltpu.async_copy(x_hbm.at[idx_v.at[0, 0]], rows.at[0], gsems[0])

    plsc.subcore_barrier()

    NIT = BPW // 4

    def stage(j, u, i):
        p = u % 2
        q = 1 - p
        t1 = (u + 1) % 4
        t2 = (u + 2) % 4

        pltpu.make_async_copy(x_hbm.at[idx_v.at[u, 0]],
                              rows.at[p], gsems[p]).wait()

        def step2():
            def wait_sc():
                pltpu.make_async_copy(
                    rows.at[q], acc.at[idx_v.at[(u + 3) % 4, 1]],
                    ssems[q]).wait()
            if u == 0:
                pl.when(i >= 1)(wait_sc)
            else:
                wait_sc()

            def pref():
                issue_idx(j + 2, t2)
            if u < 2:
                pref()
            else:
                pl.when(i < NIT - 1)(pref)

            wait_idx(j + 1, t1)
            pltpu.async_copy(x_hbm.at[idx_v.at[t1, 0]], rows.at[q],
                             gsems[q])
        if u == 3:
            pl.when(i < NIT - 1)(step2)
        else:
            step2()

        def mbody(h, _):
            w16 = w_v[u, pl.ds(h * 16, 16)] * scale
            for l in range(16):
                ws = w16[l]
                e = h * 16 + l
                for v in range(C // 16):
                    rows[p, e, pl.ds(16 * v, 16)] = (
                        rows[p, e, pl.ds(16 * v, 16)] * ws)
            return 0
        lax.fori_loop(0, K // 16, mbody, 0)

        pltpu.async_copy(rows.at[p], acc.at[idx_v.at[u, 1]],
                         ssems[p], add=True)

    def it(i, _):
        for u in range(4):
            stage(4 * i + u, u, i)
        return 0
    lax.fori_loop(0, NIT, it, 0)

    pltpu.make_async_copy(rows.at[0], acc.at[idx_v.at[2, 1]],
                          ssems[0]).wait()
    pltpu.make_async_copy(rows.at[1], acc.at[idx_v.at[3, 1]],
                          ssems[1]).wait()

    plsc.subcore_barrier()
    pltpu.sync_copy(acc.at[pl.ds(base, ROWS_PER_TILE)],
                    part_hbm.at[cid, pl.ds(base, ROWS_PER_TILE)])

    @pl.when(sid == NS - 1)
    def _():
        tail = NS * ROWS_PER_TILE
        pltpu.sync_copy(acc.at[pl.ds(tail, N - tail)],
                        part_hbm.at[cid, pl.ds(tail, N - tail)])


@functools.lru_cache(maxsize=None)
def _make_spmm(scale):
    mesh = plsc.VectorSubcoreMesh(core_axis_name="c", subcore_axis_name="s")
    return pl.kernel(
        functools.partial(_spmm_body, scale),
        out_type=jax.ShapeDtypeStruct((NC, N, C), jnp.float32),
        mesh=mesh,
        scratch_types=[
            pltpu.VMEM_SHARED((N, C), jnp.float32),
            pltpu.VMEM((4, 2, K), jnp.int32),
            pltpu.VMEM((4, K), jnp.float32),
            pltpu.VMEM((2, K, C), jnp.float32),
        ] + [pltpu.SemaphoreType.DMA] * 8,
    )


_ROWS_BLK = 1000
_GRID = N // _ROWS_BLK


def _combine_body(p0_ref, p1_ref, prev_ref, o_ref):
    o_ref[...] = p0_ref[...] + p1_ref[...] - prev_ref[...]


_combine = pl.pallas_call(
    _combine_body,
    grid=(_GRID,),
    in_specs=[pl.BlockSpec((_ROWS_BLK, C), lambda i: (i, 0))] * 3,
    out_specs=pl.BlockSpec((_ROWS_BLK, C), lambda i: (i, 0)),
    out_shape=jax.ShapeDtypeStruct((N, C), jnp.float32),
)


def _combine2_body(p0_ref, p1_ref, o_ref):
    o_ref[...] = p0_ref[...] + p1_ref[...]


_combine2 = pl.pallas_call(
    _combine2_body,
    grid=(_GRID,),
    in_specs=[pl.BlockSpec((_ROWS_BLK, C), lambda i: (i, 0))] * 2,
    out_specs=pl.BlockSpec((_ROWS_BLK, C), lambda i: (i, 0)),
    out_shape=jax.ShapeDtypeStruct((N, C), jnp.float32),
)


def _final_body(c0_ref, c1_ref, c2_ref, c3_ref, w_ref, b_ref, o_ref):
    acc = jnp.dot(c0_ref[...], w_ref[0], preferred_element_type=jnp.float32)
    acc += jnp.dot(c1_ref[...], w_ref[1], preferred_element_type=jnp.float32)
    acc += jnp.dot(c2_ref[...], w_ref[2], preferred_element_type=jnp.float32)
    acc += jnp.dot(c3_ref[...], w_ref[3], preferred_element_type=jnp.float32)
    o_ref[...] = jax.nn.relu(acc + b_ref[...])


_final = pl.pallas_call(
    _final_body,
    grid=(_GRID,),
    in_specs=[pl.BlockSpec((_ROWS_BLK, C), lambda i: (i, 0))] * 4
    + [pl.BlockSpec((4, C, C), lambda i: (0, 0, 0)),
       pl.BlockSpec((1, C), lambda i: (0, 0))],
    out_specs=pl.BlockSpec((_ROWS_BLK, C), lambda i: (i, 0)),
    out_shape=jax.ShapeDtypeStruct((N, C), jnp.float32),
)


def kernel(inputs, edge_index, edge_weight, W, b):
    x = inputs[0]
    pad = EP - E
    pad_idx = jnp.arange(pad, dtype=jnp.int32) % N
    src = jnp.concatenate([edge_index[1], pad_idx])
    dst = jnp.concatenate([edge_index[0], pad_idx])
    wgt = jnp.pad(edge_weight, (0, pad))

    _spmm_1 = _make_spmm(1.0)
    _spmm_2 = _make_spmm(2.0)

    p1 = _spmm_1(x, src, dst, wgt)
    c1 = _combine2(p1[0], p1[1])
    p2 = _spmm_2(c1, src, dst, wgt)
    c2 = _combine(p2[0], p2[1], x)
    p3 = _spmm_2(c2, src, dst, wgt)
    c3 = _combine(p3[0], p3[1], c1)

    out = _final(x, c1, c2, c3, W, b.reshape(1, C))
    return out[None]


# speedup vs baseline: 3.1844x; 1.0246x over previous
import functools

import jax
import jax.numpy as jnp
from jax import lax
from jax.experimental import pallas as pl
from jax.experimental.pallas import tpu as pltpu
from jax.experimental.pallas import tpu_sc as plsc

N = 10000
E = 320000
C = 128
K = 128
NC = 2
NS = 16
NW = NC * NS
BPW = 80
EP = NW * BPW * K
NSLOT = 4
ROWS_PER_TILE = 624


def _spmm_body(scale, x_hbm, src_hbm, dst_hbm, w_hbm, part_hbm,
               acc, idx_v, w_v, rows,
               gs0, gs1, ss0, ss1, is0, is1, is2, is3):
    gsems = (gs0, gs1)
    ssems = (ss0, ss1)
    isems = (is0, is1, is2, is3)
    cid = lax.axis_index("c")
    sid = lax.axis_index("s")
    wid = sid * NC + cid
    ebase = wid * (BPW * K)

    def issue_idx(blk, slot):
        off = ebase + blk * K
        pltpu.async_copy(src_hbm.at[pl.ds(off, K)], idx_v.at[slot, 0],
                         isems[slot])
        pltpu.async_copy(dst_hbm.at[pl.ds(off, K)], idx_v.at[slot, 1],
                         isems[slot])
        pltpu.async_copy(w_hbm.at[pl.ds(off, K)], w_v.at[slot], isems[slot])

    def wait_idx(blk, slot):
        off = ebase + blk * K
        pltpu.make_async_copy(src_hbm.at[pl.ds(off, K)], idx_v.at[slot, 0],
                              isems[slot]).wait()
        pltpu.make_async_copy(dst_hbm.at[pl.ds(off, K)], idx_v.at[slot, 1],
                              isems[slot]).wait()
        pltpu.make_async_copy(w_hbm.at[pl.ds(off, K)], w_v.at[slot],
                              isems[slot]).wait()

    issue_idx(0, 0)
    wait_idx(0, 0)
    issue_idx(1, 1)

    def zbody(r, _):
        for v in range(C // 16):
            rows[0, r, pl.ds(16 * v, 16)] = jnp.zeros((16,), jnp.float32)
        return 0
    lax.fori_loop(0, K, zbody, 0)
    base = sid * ROWS_PER_TILE
    for q in range(ROWS_PER_TILE // K):
        pltpu.sync_copy(rows.at[0], acc.at[pl.ds(base + q * K, K)])
    rem = ROWS_PER_TILE % K
    pltpu.sync_copy(rows.at[0, pl.ds(0, rem)],
                    acc.at[pl.ds(base + ROWS_PER_TILE - rem, rem)])

    @pl.when(sid == NS - 1)
    def _():
        tail = NS * ROWS_PER_TILE
        pltpu.sync_copy(rows.at[0, pl.ds(0, 16)],
                        acc.at[pl.ds(tail, N - tail)])

    pltpu.async_copy(x_hbm.at[idx_v.at[0, 0]], rows.at[0], gsems[0])

    plsc.subcore_barrier()

    NIT = BPW // 4

    def stage(j, u, i):
        p = u % 2
        q = 1 - p
        t1 = (u + 1) % 4
        t2 = (u + 2) % 4

        def step2():
            def wait_sc():
                pltpu.make_async_copy(
                    rows.at[q], acc.at[idx_v.at[(u + 3) % 4, 1]],
                    ssems[q]).wait()
            if u == 0:
                pl.when(i >= 1)(wait_sc)
            else:
                wait_sc()

            def pref():
                issue_idx(j + 2, t2)
            if u < 2:
                pref()
            else:
                pl.when(i < NIT - 1)(pref)

            wait_idx(j + 1, t1)
            pltpu.async_copy(x_hbm.at[idx_v.at[t1, 0]], rows.at[q],
                             gsems[q])
        if u == 3:
            pl.when(i < NIT - 1)(step2)
        else:
            step2()

        pltpu.make_async_copy(x_hbm.at[idx_v.at[u, 0]],
                              rows.at[p], gsems[p]).wait()

        def mbody(h, _):
            w16 = w_v[u, pl.ds(h * 16, 16)] * scale
            for l in range(16):
                ws = w16[l]
                e = h * 16 + l
                for v in range(C // 16):
                    rows[p, e, pl.ds(16 * v, 16)] = (
                        rows[p, e, pl.ds(16 * v, 16)] * ws)
            return 0
        lax.fori_loop(0, K // 16, mbody, 0)

        pltpu.async_copy(rows.at[p], acc.at[idx_v.at[u, 1]],
                         ssems[p], add=True)

    def it(i, _):
        for u in range(4):
            stage(4 * i + u, u, i)
        return 0
    lax.fori_loop(0, NIT, it, 0)

    pltpu.make_async_copy(rows.at[0], acc.at[idx_v.at[2, 1]],
                          ssems[0]).wait()
    pltpu.make_async_copy(rows.at[1], acc.at[idx_v.at[3, 1]],
                          ssems[1]).wait()

    plsc.subcore_barrier()
    pltpu.sync_copy(acc.at[pl.ds(base, ROWS_PER_TILE)],
                    part_hbm.at[cid, pl.ds(base, ROWS_PER_TILE)])

    @pl.when(sid == NS - 1)
    def _():
        tail = NS * ROWS_PER_TILE
        pltpu.sync_copy(acc.at[pl.ds(tail, N - tail)],
                        part_hbm.at[cid, pl.ds(tail, N - tail)])


@functools.lru_cache(maxsize=None)
def _make_spmm(scale):
    mesh = plsc.VectorSubcoreMesh(core_axis_name="c", subcore_axis_name="s")
    return pl.kernel(
        functools.partial(_spmm_body, scale),
        out_type=jax.ShapeDtypeStruct((NC, N, C), jnp.float32),
        mesh=mesh,
        scratch_types=[
            pltpu.VMEM_SHARED((N, C), jnp.float32),
            pltpu.VMEM((4, 2, K), jnp.int32),
            pltpu.VMEM((4, K), jnp.float32),
            pltpu.VMEM((2, K, C), jnp.float32),
        ] + [pltpu.SemaphoreType.DMA] * 8,
    )


_ROWS_BLK = 1000
_GRID = N // _ROWS_BLK


def _combine_body(p0_ref, p1_ref, prev_ref, o_ref):
    o_ref[...] = p0_ref[...] + p1_ref[...] - prev_ref[...]


_combine = pl.pallas_call(
    _combine_body,
    grid=(_GRID,),
    in_specs=[pl.BlockSpec((_ROWS_BLK, C), lambda i: (i, 0))] * 3,
    out_specs=pl.BlockSpec((_ROWS_BLK, C), lambda i: (i, 0)),
    out_shape=jax.ShapeDtypeStruct((N, C), jnp.float32),
)


def _combine2_body(p0_ref, p1_ref, o_ref):
    o_ref[...] = p0_ref[...] + p1_ref[...]


_combine2 = pl.pallas_call(
    _combine2_body,
    grid=(_GRID,),
    in_specs=[pl.BlockSpec((_ROWS_BLK, C), lambda i: (i, 0))] * 2,
    out_specs=pl.BlockSpec((_ROWS_BLK, C), lambda i: (i, 0)),
    out_shape=jax.ShapeDtypeStruct((N, C), jnp.float32),
)


def _final_body(c0_ref, c1_ref, c2_ref, c3_ref, w_ref, b_ref, o_ref):
    acc = jnp.dot(c0_ref[...], w_ref[0], preferred_element_type=jnp.float32)
    acc += jnp.dot(c1_ref[...], w_ref[1], preferred_element_type=jnp.float32)
    acc += jnp.dot(c2_ref[...], w_ref[2], preferred_element_type=jnp.float32)
    acc += jnp.dot(c3_ref[...], w_ref[3], preferred_element_type=jnp.float32)
    o_ref[...] = jax.nn.relu(acc + b_ref[...])


_final = pl.pallas_call(
    _final_body,
    grid=(_GRID,),
    in_specs=[pl.BlockSpec((_ROWS_BLK, C), lambda i: (i, 0))] * 4
    + [pl.BlockSpec((4, C, C), lambda i: (0, 0, 0)),
       pl.BlockSpec((1, C), lambda i: (0, 0))],
    out_specs=pl.BlockSpec((_ROWS_BLK, C), lambda i: (i, 0)),
    out_shape=jax.ShapeDtypeStruct((N, C), jnp.float32),
)


def kernel(inputs, edge_index, edge_weight, W, b):
    x = inputs[0]
    pad = EP - E
    pad_idx = jnp.arange(pad, dtype=jnp.int32) % N
    src = jnp.concatenate([edge_index[1], pad_idx])
    dst = jnp.concatenate([edge_index[0], pad_idx])
    wgt = jnp.pad(edge_weight, (0, pad))

    _spmm_1 = _make_spmm(1.0)
    _spmm_2 = _make_spmm(2.0)

    p1 = _spmm_1(x, src, dst, wgt)
    c1 = _combine2(p1[0], p1[1])
    p2 = _spmm_2(c1, src, dst, wgt)
    c2 = _combine(p2[0], p2[1], x)
    p3 = _spmm_2(c2, src, dst, wgt)
    c3 = _combine(p3[0], p3[1], c1)

    out = _final(x, c1, c2, c3, W, b.reshape(1, C))
    return out[None]


# speedup vs baseline: 3.2207x; 1.0114x over previous
import functools

import jax
import jax.numpy as jnp
from jax import lax
from jax.experimental import pallas as pl
from jax.experimental.pallas import tpu as pltpu
from jax.experimental.pallas import tpu_sc as plsc

N = 10000
E = 320000
C = 128
K = 128
NC = 2
NS = 16
NW = NC * NS
BPW = 80
EP = NW * BPW * K
NSLOT = 4
ROWS_PER_TILE = 624


def _spmm_body(scale, x_hbm, src_hbm, dst_hbm, w_hbm, part_hbm,
               acc, idx_v, w_v, rows,
               gs0, gs1, ss0, ss1, is0, is1, is2, is3):
    gsems = (gs0, gs1)
    ssems = (ss0, ss1)
    isems = (is0, is1, is2, is3)
    cid = lax.axis_index("c")
    sid = lax.axis_index("s")
    wid = sid * NC + cid
    ebase = wid * (BPW * K)

    def issue_idx(blk, slot):
        off = ebase + blk * K
        pltpu.async_copy(src_hbm.at[pl.ds(off, K)], idx_v.at[slot, 0],
                         isems[slot])
        pltpu.async_copy(dst_hbm.at[pl.ds(off, K)], idx_v.at[slot, 1],
                         isems[slot])
        pltpu.async_copy(w_hbm.at[pl.ds(off, K)], w_v.at[slot], isems[slot])

    def wait_idx(blk, slot):
        off = ebase + blk * K
        pltpu.make_async_copy(src_hbm.at[pl.ds(off, K)], idx_v.at[slot, 0],
                              isems[slot]).wait()
        pltpu.make_async_copy(dst_hbm.at[pl.ds(off, K)], idx_v.at[slot, 1],
                              isems[slot]).wait()
        pltpu.make_async_copy(w_hbm.at[pl.ds(off, K)], w_v.at[slot],
                              isems[slot]).wait()

    issue_idx(0, 0)
    wait_idx(0, 0)
    issue_idx(1, 1)

    def zbody(r, _):
        for v in range(C // 16):
            rows[0, r, pl.ds(16 * v, 16)] = jnp.zeros((16,), jnp.float32)
        return 0
    lax.fori_loop(0, K, zbody, 0)
    base = sid * ROWS_PER_TILE
    for q in range(ROWS_PER_TILE // K):
        pltpu.sync_copy(rows.at[0], acc.at[pl.ds(base + q * K, K)])
    rem = ROWS_PER_TILE % K
    pltpu.sync_copy(rows.at[0, pl.ds(0, rem)],
                    acc.at[pl.ds(base + ROWS_PER_TILE - rem, rem)])

    @pl.when(sid == NS - 1)
    def _():
        tail = NS * ROWS_PER_TILE
        pltpu.sync_copy(rows.at[0, pl.ds(0, 16)],
                        acc.at[pl.ds(tail, N - tail)])

    pltpu.async_copy(x_hbm.at[idx_v.at[0, 0]], rows.at[0], gsems[0])

    plsc.subcore_barrier()

    NIT = BPW // 4

    def stage(j, u, i):
        p = u % 2
        q = 1 - p
        t1 = (u + 1) % 4
        t2 = (u + 2) % 4

        def step2():
            def wait_sc():
                pltpu.make_async_copy(
                    rows.at[q], acc.at[idx_v.at[(u + 3) % 4, 1]],
                    ssems[q]).wait()
            if u == 0:
                pl.when(i >= 1)(wait_sc)
            else:
                wait_sc()

            def pref():
                issue_idx(j + 2, t2)
            if u < 2:
                pref()
            else:
                pl.when(i < NIT - 1)(pref)

            wait_idx(j + 1, t1)
            pltpu.async_copy(x_hbm.at[idx_v.at[t1, 0]], rows.at[q],
                             gsems[q])
        if u == 3:
            pl.when(i < NIT - 1)(step2)
        else:
            step2()

        pltpu.make_async_copy(x_hbm.at[idx_v.at[u, 0]],
                              rows.at[p], gsems[p]).wait()

        def mbody(h, _):
            w16 = w_v[u, pl.ds(h * 16, 16)] * scale
            for l in range(16):
                ws = w16[l]
                e = h * 16 + l
                for v in range(C // 16):
                    rows[p, e, pl.ds(16 * v, 16)] = (
                        rows[p, e, pl.ds(16 * v, 16)] * ws)
            return 0
        lax.fori_loop(0, K // 16, mbody, 0)

        pltpu.async_copy(rows.at[p], acc.at[idx_v.at[u, 1]],
                         ssems[p], add=True)

    def it(i, _):
        for u in range(4):
            stage(4 * i + u, u, i)
        return 0
    lax.fori_loop(0, NIT, it, 0)

    pltpu.make_async_copy(rows.at[0], acc.at[idx_v.at[2, 1]],
                          ssems[0]).wait()
    pltpu.make_async_copy(rows.at[1], acc.at[idx_v.at[3, 1]],
                          ssems[1]).wait()

    plsc.subcore_barrier()
    pltpu.sync_copy(acc.at[pl.ds(base, ROWS_PER_TILE)],
                    part_hbm.at[cid, pl.ds(base, ROWS_PER_TILE)])

    @pl.when(sid == NS - 1)
    def _():
        tail = NS * ROWS_PER_TILE
        pltpu.sync_copy(acc.at[pl.ds(tail, N - tail)],
                        part_hbm.at[cid, pl.ds(tail, N - tail)])


@functools.lru_cache(maxsize=None)
def _make_spmm(scale):
    mesh = plsc.VectorSubcoreMesh(core_axis_name="c", subcore_axis_name="s")
    return pl.kernel(
        functools.partial(_spmm_body, scale),
        out_type=jax.ShapeDtypeStruct((NC, N, C), jnp.float32),
        mesh=mesh,
        scratch_types=[
            pltpu.VMEM_SHARED((N, C), jnp.float32),
            pltpu.VMEM((4, 2, K), jnp.int32),
            pltpu.VMEM((4, K), jnp.float32),
            pltpu.VMEM((2, K, C), jnp.float32),
        ] + [pltpu.SemaphoreType.DMA] * 8,
    )


_ROWS_BLK = 1000
_GRID = N // _ROWS_BLK
_BS = pl.BlockSpec((_ROWS_BLK, C), lambda i: (i, 0))
_WS = pl.BlockSpec((4, C, C), lambda i: (0, 0, 0))


def _k1_body(p0_ref, p1_ref, x_ref, w_ref, c1_ref, a0_ref):
    c1 = p0_ref[...] + p1_ref[...]
    c1_ref[...] = c1
    a0_ref[...] = (
        jnp.dot(x_ref[...], w_ref[0], preferred_element_type=jnp.float32)
        + jnp.dot(c1, w_ref[1], preferred_element_type=jnp.float32))


_k1 = pl.pallas_call(
    _k1_body,
    grid=(_GRID,),
    in_specs=[_BS, _BS, _BS, _WS],
    out_specs=[_BS, _BS],
    out_shape=[jax.ShapeDtypeStruct((N, C), jnp.float32)] * 2,
)


def _k2_body(p0_ref, p1_ref, x_ref, a0_ref, w_ref, c2_ref, a1_ref):
    c2 = p0_ref[...] + p1_ref[...] - x_ref[...]
    c2_ref[...] = c2
    a1_ref[...] = a0_ref[...] + jnp.dot(
        c2, w_ref[2], preferred_element_type=jnp.float32)


_k2 = pl.pallas_call(
    _k2_body,
    grid=(_GRID,),
    in_specs=[_BS, _BS, _BS, _BS, _WS],
    out_specs=[_BS, _BS],
    out_shape=[jax.ShapeDtypeStruct((N, C), jnp.float32)] * 2,
)


def _k3_body(p0_ref, p1_ref, c1_ref, a1_ref, w_ref, b_ref, o_ref):
    c3 = p0_ref[...] + p1_ref[...] - c1_ref[...]
    o_ref[...] = jax.nn.relu(
        a1_ref[...]
        + jnp.dot(c3, w_ref[3], preferred_element_type=jnp.float32)
        + b_ref[...])


_k3 = pl.pallas_call(
    _k3_body,
    grid=(_GRID,),
    in_specs=[_BS, _BS, _BS, _BS, _WS,
              pl.BlockSpec((1, C), lambda i: (0, 0))],
    out_specs=_BS,
    out_shape=jax.ShapeDtypeStruct((N, C), jnp.float32),
)


def kernel(inputs, edge_index, edge_weight, W, b):
    x = inputs[0]
    pad = EP - E
    pad_idx = jnp.arange(pad, dtype=jnp.int32) % N
    src = jnp.concatenate([edge_index[1], pad_idx])
    dst = jnp.concatenate([edge_index[0], pad_idx])
    wgt = jnp.pad(edge_weight, (0, pad))

    _spmm_1 = _make_spmm(1.0)
    _spmm_2 = _make_spmm(2.0)

    p1 = _spmm_1(x, src, dst, wgt)
    c1, a0 = _k1(p1[0], p1[1], x, W)
    p2 = _spmm_2(c1, src, dst, wgt)
    c2, a1 = _k2(p2[0], p2[1], x, a0, W)
    p3 = _spmm_2(c2, src, dst, wgt)
    out = _k3(p3[0], p3[1], c1, a1, W, b.reshape(1, C))
    return out[None]


# speedup vs baseline: 3.2376x; 1.0053x over previous
import functools

import jax
import jax.numpy as jnp
from jax import lax
from jax.experimental import pallas as pl
from jax.experimental.pallas import tpu as pltpu
from jax.experimental.pallas import tpu_sc as plsc

N = 10000
E = 320000
C = 128
K = 128
NC = 2
NS = 16
NW = NC * NS
BPW = 80
EP = NW * BPW * K
NSLOT = 4
ROWS_PER_TILE = 624


def _spmm_body(scale, x_hbm, src_hbm, dst_hbm, w_hbm, part_hbm,
               acc, idx_v, w_v, rows,
               gs0, gs1, ss0, ss1, is0, is1, is2, is3):
    gsems = (gs0, gs1)
    ssems = (ss0, ss1)
    isems = (is0, is1, is2, is3)
    cid = lax.axis_index("c")
    sid = lax.axis_index("s")
    wid = sid * NC + cid
    ebase = wid * (BPW * K)

    def issue_idx(blk, slot):
        off = ebase + blk * K
        pltpu.async_copy(src_hbm.at[pl.ds(off, K)], idx_v.at[slot, 0],
                         isems[slot])
        pltpu.async_copy(dst_hbm.at[pl.ds(off, K)], idx_v.at[slot, 1],
                         isems[slot])
        pltpu.async_copy(w_hbm.at[pl.ds(off, K)], w_v.at[slot], isems[slot])

    def wait_idx(blk, slot):
        off = ebase + blk * K
        pltpu.make_async_copy(src_hbm.at[pl.ds(off, K)], idx_v.at[slot, 0],
                              isems[slot]).wait()
        pltpu.make_async_copy(dst_hbm.at[pl.ds(off, K)], idx_v.at[slot, 1],
                              isems[slot]).wait()
        pltpu.make_async_copy(w_hbm.at[pl.ds(off, K)], w_v.at[slot],
                              isems[slot]).wait()

    issue_idx(0, 0)
    wait_idx(0, 0)
    pltpu.async_copy(x_hbm.at[idx_v.at[0, 0]], rows.at[0], gsems[0])
    issue_idx(1, 1)

    def zbody(r, _):
        for v in range(C // 16):
            rows[1, r, pl.ds(16 * v, 16)] = jnp.zeros((16,), jnp.float32)
        return 0
    lax.fori_loop(0, K, zbody, 0)
    base = sid * ROWS_PER_TILE
    for q in range(ROWS_PER_TILE // K):
        pltpu.async_copy(rows.at[1], acc.at[pl.ds(base + q * K, K)], ss0)
    rem = ROWS_PER_TILE % K
    pltpu.async_copy(rows.at[1, pl.ds(0, rem)],
                     acc.at[pl.ds(base + ROWS_PER_TILE - rem, rem)], ss0)

    @pl.when(sid == NS - 1)
    def _():
        tail = NS * ROWS_PER_TILE
        pltpu.sync_copy(rows.at[1, pl.ds(0, 16)],
                        acc.at[pl.ds(tail, N - tail)])

    for q in range(ROWS_PER_TILE // K):
        pltpu.make_async_copy(rows.at[1], acc.at[pl.ds(base + q * K, K)],
                              ss0).wait()
    pltpu.make_async_copy(rows.at[1, pl.ds(0, rem)],
                          acc.at[pl.ds(base + ROWS_PER_TILE - rem, rem)],
                          ss0).wait()

    plsc.subcore_barrier()

    NIT = BPW // 4

    def stage(j, u, i):
        p = u % 2
        q = 1 - p
        t1 = (u + 1) % 4
        t2 = (u + 2) % 4

        def step2():
            def wait_sc():
                pltpu.make_async_copy(
                    rows.at[q], acc.at[idx_v.at[(u + 3) % 4, 1]],
                    ssems[q]).wait()
            if u == 0:
                pl.when(i >= 1)(wait_sc)
            else:
                wait_sc()

            def pref():
                issue_idx(j + 2, t2)
            if u < 2:
                pref()
            else:
                pl.when(i < NIT - 1)(pref)

            wait_idx(j + 1, t1)
            pltpu.async_copy(x_hbm.at[idx_v.at[t1, 0]], rows.at[q],
                             gsems[q])
        if u == 3:
            pl.when(i < NIT - 1)(step2)
        else:
            step2()

        pltpu.make_async_copy(x_hbm.at[idx_v.at[u, 0]],
                              rows.at[p], gsems[p]).wait()

        def mbody(h, _):
            w16 = w_v[u, pl.ds(h * 16, 16)] * scale
            for l in range(16):
                ws = w16[l]
                e = h * 16 + l
                for v in range(C // 16):
                    rows[p, e, pl.ds(16 * v, 16)] = (
                        rows[p, e, pl.ds(16 * v, 16)] * ws)
            return 0
        lax.fori_loop(0, K // 16, mbody, 0)

        pltpu.async_copy(rows.at[p], acc.at[idx_v.at[u, 1]],
                         ssems[p], add=True)

    def it(i, _):
        for u in range(4):
            stage(4 * i + u, u, i)
        return 0
    lax.fori_loop(0, NIT, it, 0)

    pltpu.make_async_copy(rows.at[0], acc.at[idx_v.at[2, 1]],
                          ssems[0]).wait()
    pltpu.make_async_copy(rows.at[1], acc.at[idx_v.at[3, 1]],
                          ssems[1]).wait()

    plsc.subcore_barrier()
    pltpu.sync_copy(acc.at[pl.ds(base, ROWS_PER_TILE)],
                    part_hbm.at[cid, pl.ds(base, ROWS_PER_TILE)])

    @pl.when(sid == NS - 1)
    def _():
        tail = NS * ROWS_PER_TILE
        pltpu.sync_copy(acc.at[pl.ds(tail, N - tail)],
                        part_hbm.at[cid, pl.ds(tail, N - tail)])


@functools.lru_cache(maxsize=None)
def _make_spmm(scale):
    mesh = plsc.VectorSubcoreMesh(core_axis_name="c", subcore_axis_name="s")
    return pl.kernel(
        functools.partial(_spmm_body, scale),
        out_type=jax.ShapeDtypeStruct((NC, N, C), jnp.float32),
        mesh=mesh,
        scratch_types=[
            pltpu.VMEM_SHARED((N, C), jnp.float32),
            pltpu.VMEM((4, 2, K), jnp.int32),
            pltpu.VMEM((4, K), jnp.float32),
            pltpu.VMEM((2, K, C), jnp.float32),
        ] + [pltpu.SemaphoreType.DMA] * 8,
    )


_ROWS_BLK = 1000
_GRID = N // _ROWS_BLK
_BS = pl.BlockSpec((_ROWS_BLK, C), lambda i: (i, 0))
_WS = pl.BlockSpec((4, C, C), lambda i: (0, 0, 0))


def _k1_body(p0_ref, p1_ref, x_ref, w_ref, c1_ref, a0_ref):
    c1 = p0_ref[...] + p1_ref[...]
    c1_ref[...] = c1
    a0_ref[...] = (
        jnp.dot(x_ref[...], w_ref[0], preferred_element_type=jnp.float32)
        + jnp.dot(c1, w_ref[1], preferred_element_type=jnp.float32))


_k1 = pl.pallas_call(
    _k1_body,
    grid=(_GRID,),
    in_specs=[_BS, _BS, _BS, _WS],
    out_specs=[_BS, _BS],
    out_shape=[jax.ShapeDtypeStruct((N, C), jnp.float32)] * 2,
)


def _k2_body(p0_ref, p1_ref, x_ref, a0_ref, w_ref, c2_ref, a1_ref):
    c2 = p0_ref[...] + p1_ref[...] - x_ref[...]
    c2_ref[...] = c2
    a1_ref[...] = a0_ref[...] + jnp.dot(
        c2, w_ref[2], preferred_element_type=jnp.float32)


_k2 = pl.pallas_call(
    _k2_body,
    grid=(_GRID,),
    in_specs=[_BS, _BS, _BS, _BS, _WS],
    out_specs=[_BS, _BS],
    out_shape=[jax.ShapeDtypeStruct((N, C), jnp.float32)] * 2,
)


def _k3_body(p0_ref, p1_ref, c1_ref, a1_ref, w_ref, b_ref, o_ref):
    c3 = p0_ref[...] + p1_ref[...] - c1_ref[...]
    o_ref[...] = jax.nn.relu(
        a1_ref[...]
        + jnp.dot(c3, w_ref[3], preferred_element_type=jnp.float32)
        + b_ref[...])


_k3 = pl.pallas_call(
    _k3_body,
    grid=(_GRID,),
    in_specs=[_BS, _BS, _BS, _BS, _WS,
              pl.BlockSpec((1, C), lambda i: (0, 0))],
    out_specs=_BS,
    out_shape=jax.ShapeDtypeStruct((N, C), jnp.float32),
)


def kernel(inputs, edge_index, edge_weight, W, b):
    x = inputs[0]
    pad = EP - E
    pad_idx = jnp.arange(pad, dtype=jnp.int32) % N
    src = jnp.concatenate([edge_index[1], pad_idx])
    dst = jnp.concatenate([edge_index[0], pad_idx])
    wgt = jnp.pad(edge_weight, (0, pad))

    _spmm_1 = _make_spmm(1.0)
    _spmm_2 = _make_spmm(2.0)

    p1 = _spmm_1(x, src, dst, wgt)
    c1, a0 = _k1(p1[0], p1[1], x, W)
    p2 = _spmm_2(c1, src, dst, wgt)
    c2, a1 = _k2(p2[0], p2[1], x, a0, W)
    p3 = _spmm_2(c2, src, dst, wgt)
    out = _k3(p3[0], p3[1], c1, a1, W, b.reshape(1, C))
    return out[None]


# speedup vs baseline: 3.2534x; 1.0049x over previous
import functools

import jax
import jax.numpy as jnp
from jax import lax
from jax.experimental import pallas as pl
from jax.experimental.pallas import tpu as pltpu
from jax.experimental.pallas import tpu_sc as plsc

N = 10000
E = 320000
C = 128
K = 128
NC = 2
NS = 16
NW = NC * NS
BPW = 80
EP = NW * BPW * K
ROWS_PER_TILE = 624


def _spmm_body(scale, x_hbm, src_hbm, dst_hbm, w_hbm, part_hbm,
               acc, idx_v, w_v, rows,
               gs0, gs1, ss0, ss1, is0, is1, is2, is3):
    gsems = (gs0, gs1)
    ssems = (ss0, ss1)
    isems = (is0, is1, is2, is3)
    cid = lax.axis_index("c")
    sid = lax.axis_index("s")
    wid = sid * NC + cid
    ebase = wid * (BPW * K)

    def issue_idx(blk, slot):
        off = ebase + blk * K
        pltpu.async_copy(src_hbm.at[pl.ds(off, K)], idx_v.at[slot, 0],
                         isems[slot])
        pltpu.async_copy(dst_hbm.at[pl.ds(off, K)], idx_v.at[slot, 1],
                         isems[slot])
        pltpu.async_copy(w_hbm.at[pl.ds(off, K)], w_v.at[slot], isems[slot])

    def wait_idx(blk, slot):
        off = ebase + blk * K
        pltpu.make_async_copy(src_hbm.at[pl.ds(off, K)], idx_v.at[slot, 0],
                              isems[slot]).wait()
        pltpu.make_async_copy(dst_hbm.at[pl.ds(off, K)], idx_v.at[slot, 1],
                              isems[slot]).wait()
        pltpu.make_async_copy(w_hbm.at[pl.ds(off, K)], w_v.at[slot],
                              isems[slot]).wait()

    issue_idx(0, 0)
    wait_idx(0, 0)
    pltpu.async_copy(x_hbm.at[idx_v.at[0, 0]], rows.at[0], gsems[0])
    issue_idx(1, 1)

    def zbody(r, _):
        for v in range(C // 16):
            rows[1, r, pl.ds(16 * v, 16)] = jnp.zeros((16,), jnp.float32)
        return 0
    lax.fori_loop(0, K, zbody, 0)
    base = sid * ROWS_PER_TILE
    for q in range(ROWS_PER_TILE // K):
        pltpu.async_copy(rows.at[1], acc.at[pl.ds(base + q * K, K)], ss0)
    rem = ROWS_PER_TILE % K
    pltpu.async_copy(rows.at[1, pl.ds(0, rem)],
                     acc.at[pl.ds(base + ROWS_PER_TILE - rem, rem)], ss0)

    @pl.when(sid == NS - 1)
    def _():
        tail = NS * ROWS_PER_TILE
        pltpu.sync_copy(rows.at[1, pl.ds(0, 16)],
                        acc.at[pl.ds(tail, N - tail)])

    for q in range(ROWS_PER_TILE // K):
        pltpu.make_async_copy(rows.at[1], acc.at[pl.ds(base + q * K, K)],
                              ss0).wait()
    pltpu.make_async_copy(rows.at[1, pl.ds(0, rem)],
                          acc.at[pl.ds(base + ROWS_PER_TILE - rem, rem)],
                          ss0).wait()

    plsc.subcore_barrier()

    NIT = BPW // 4

    def stage(j, u, i):
        p = u % 2
        q = 1 - p
        t1 = (u + 1) % 4
        t2 = (u + 2) % 4

        def step2():
            def wait_sc():
                pltpu.make_async_copy(
                    rows.at[q], acc.at[idx_v.at[(u + 3) % 4, 1]],
                    ssems[q]).wait()
            if u == 0:
                pl.when(i >= 1)(wait_sc)
            else:
                wait_sc()

            def pref():
                issue_idx(j + 2, t2)
            if u < 2:
                pref()
            else:
                pl.when(i < NIT - 1)(pref)

            wait_idx(j + 1, t1)
            pltpu.async_copy(x_hbm.at[idx_v.at[t1, 0]], rows.at[q],
                             gsems[q])
        if u == 3:
            pl.when(i < NIT - 1)(step2)
        else:
            step2()

        pltpu.make_async_copy(x_hbm.at[idx_v.at[u, 0]],
                              rows.at[p], gsems[p]).wait()

        def mbody(h, _):
            w16 = w_v[u, pl.ds(h * 16, 16)] * scale
            for l in range(16):
                ws = w16[l]
                e = h * 16 + l
                for v in range(C // 16):
                    rows[p, e, pl.ds(16 * v, 16)] = (
                        rows[p, e, pl.ds(16 * v, 16)] * ws)
            return 0
        lax.fori_loop(0, K // 16, mbody, 0)

        pltpu.async_copy(rows.at[p], acc.at[idx_v.at[u, 1]],
                         ssems[p], add=True)

    def it(i, _):
        for u in range(4):
            stage(4 * i + u, u, i)
        return 0
    lax.fori_loop(0, NIT, it, 0)

    pltpu.make_async_copy(rows.at[0], acc.at[idx_v.at[2, 1]],
                          ssems[0]).wait()
    pltpu.make_async_copy(rows.at[1], acc.at[idx_v.at[3, 1]],
                          ssems[1]).wait()

    plsc.subcore_barrier()
    pltpu.sync_copy(acc.at[pl.ds(base, ROWS_PER_TILE)],
                    part_hbm.at[cid, pl.ds(base, ROWS_PER_TILE)])

    @pl.when(sid == NS - 1)
    def _():
        tail = NS * ROWS_PER_TILE
        pltpu.sync_copy(acc.at[pl.ds(tail, N - tail)],
                        part_hbm.at[cid, pl.ds(tail, N - tail)])


@functools.lru_cache(maxsize=None)
def _make_spmm(scale):
    mesh = plsc.VectorSubcoreMesh(core_axis_name="c", subcore_axis_name="s")
    return pl.kernel(
        functools.partial(_spmm_body, scale),
        out_type=jax.ShapeDtypeStruct((NC, N, C), jnp.float32),
        mesh=mesh,
        scratch_types=[
            pltpu.VMEM_SHARED((N, C), jnp.float32),
            pltpu.VMEM((4, 2, K), jnp.int32),
            pltpu.VMEM((4, K), jnp.float32),
            pltpu.VMEM((2, K, C), jnp.float32),
        ] + [pltpu.SemaphoreType.DMA] * 8,
    )


_ROWS_BLK = 1000
_GRID = N // _ROWS_BLK
_BS = pl.BlockSpec((_ROWS_BLK, C), lambda i: (i, 0))
_WS = pl.BlockSpec((4, C, C), lambda i: (0, 0, 0))


def _k1_body(p0_ref, p1_ref, x_ref, w_ref, c1_ref, a0_ref):
    c1 = p0_ref[...] + p1_ref[...]
    c1_ref[...] = c1
    a0_ref[...] = (
        jnp.dot(x_ref[...], w_ref[0], preferred_element_type=jnp.float32)
        + jnp.dot(c1, w_ref[1], preferred_element_type=jnp.float32))


_k1 = pl.pallas_call(
    _k1_body,
    grid=(_GRID,),
    in_specs=[_BS, _BS, _BS, _WS],
    out_specs=[_BS, _BS],
    out_shape=[jax.ShapeDtypeStruct((N, C), jnp.float32)] * 2,
)


def _k2_body(p0_ref, p1_ref, x_ref, a0_ref, w_ref, c2_ref, a1_ref):
    c2 = p0_ref[...] + p1_ref[...] - x_ref[...]
    c2_ref[...] = c2
    a1_ref[...] = a0_ref[...] + jnp.dot(
        c2, w_ref[2], preferred_element_type=jnp.float32)


_k2 = pl.pallas_call(
    _k2_body,
    grid=(_GRID,),
    in_specs=[_BS, _BS, _BS, _BS, _WS],
    out_specs=[_BS, _BS],
    out_shape=[jax.ShapeDtypeStruct((N, C), jnp.float32)] * 2,
)


def _k3_body(p0_ref, p1_ref, c1_ref, a1_ref, w_ref, b_ref, o_ref):
    c3 = p0_ref[...] + p1_ref[...] - c1_ref[...]
    o_ref[...] = jax.nn.relu(
        a1_ref[...]
        + jnp.dot(c3, w_ref[3], preferred_element_type=jnp.float32)
        + b_ref[...])


_k3 = pl.pallas_call(
    _k3_body,
    grid=(_GRID,),
    in_specs=[_BS, _BS, _BS, _BS, _WS,
              pl.BlockSpec((1, C), lambda i: (0, 0))],
    out_specs=_BS,
    out_shape=jax.ShapeDtypeStruct((N, C), jnp.float32),
)


def kernel(inputs, edge_index, edge_weight, W, b):
    x = inputs[0]
    pad = EP - E
    pad_idx = jnp.arange(pad, dtype=jnp.int32) % N
    src = jnp.concatenate([edge_index[1], pad_idx])
    dst = jnp.concatenate([edge_index[0], pad_idx])
    wgt = jnp.pad(edge_weight, (0, pad))

    _spmm_1 = _make_spmm(1.0)
    _spmm_2 = _make_spmm(2.0)

    p1 = _spmm_1(x, src, dst, wgt)
    c1, a0 = _k1(p1[0], p1[1], x, W)
    p2 = _spmm_2(c1, src, dst, wgt)
    c2, a1 = _k2(p2[0], p2[1], x, a0, W)
    p3 = _spmm_2(c2, src, dst, wgt)
    out = _k3(p3[0], p3[1], c1, a1, W, b.reshape(1, C))
    return out[None]
